# Initial kernel scaffold; baseline (speedup 1.0000x reference)
import functools

import jax
import jax.numpy as jnp
from jax import lax
from jax.experimental import pallas as pl
from jax.experimental.pallas import tpu as pltpu
from jax.experimental.pallas import tpu_sc as plsc

N_CLASSES = 1000
BATCH = 16384
_L = 16

_NC = 2
_NS = 16
_NW = _NC * _NS
_ROWS_PER_W = BATCH // _NW
_C = 16
_NCHUNK = _ROWS_PER_W // _C


def _onehot_body(p_hbm, cm_hbm, out_hbm, p_v, buf):
    del cm_hbm
    wid = lax.axis_index("s") * _NC + lax.axis_index("c")
    base = wid * _ROWS_PER_W
    pltpu.sync_copy(p_hbm.at[pl.ds(base, _ROWS_PER_W)], p_v)

    zeros16 = jnp.zeros((_L,), jnp.float32)
    ones16 = jnp.ones((_L,), jnp.float32)
    rows = lax.iota(jnp.int32, _L)

    def zrow(r, carry):
        for j in range(0, N_CLASSES, _L):
            off = min(j, N_CLASSES - _L)
            buf[r, pl.ds(off, _L)] = zeros16
        return carry

    lax.fori_loop(0, _C, zrow, 0)

    def chunk(g, carry):
        off = g * _C
        cols = p_v[pl.ds(off, _L)]
        plsc.store_scatter(buf, [rows, cols], ones16)
        pltpu.sync_copy(buf, out_hbm.at[pl.ds(base + off, _C)])
        plsc.store_scatter(buf, [rows, cols], zeros16)
        return carry

    lax.fori_loop(0, _NCHUNK, chunk, 0)


def kernel(p, class_matrix):
    mesh = plsc.VectorSubcoreMesh(core_axis_name="c", subcore_axis_name="s")
    run = functools.partial(
        pl.kernel,
        mesh=mesh,
        out_type=jax.ShapeDtypeStruct((BATCH, N_CLASSES), jnp.float32),
        scratch_types=[
            pltpu.VMEM((_ROWS_PER_W,), jnp.int32),
            pltpu.VMEM((_C, N_CLASSES), jnp.float32),
        ],
    )(_onehot_body)
    return run(p.astype(jnp.int32), class_matrix)


# speedup vs baseline: 1.1034x; 1.1034x over previous
import functools

import jax
import jax.numpy as jnp
from jax import lax
from jax.experimental import pallas as pl
from jax.experimental.pallas import tpu as pltpu
from jax.experimental.pallas import tpu_sc as plsc

N_CLASSES = 1000
BATCH = 16384
_L = 16

_NC = 2
_NS = 16
_NW = _NC * _NS
_ROWS_PER_W = BATCH // _NW
_C = 16
_NCHUNK = _ROWS_PER_W // _C
_BUF = _C * N_CLASSES


def _onehot_body(p_hbm, cm_hbm, out_hbm, p_v, buf):
    del cm_hbm
    wid = lax.axis_index("s") * _NC + lax.axis_index("c")
    base = wid * _ROWS_PER_W
    pltpu.sync_copy(p_hbm.at[pl.ds(base, _ROWS_PER_W)], p_v)

    zeros16 = jnp.zeros((_L,), jnp.float32)
    ones16 = jnp.ones((_L,), jnp.float32)
    row_off = lax.iota(jnp.int32, _L) * N_CLASSES

    def zero(i, carry):
        for u in range(8):
            buf[pl.ds((i * 8 + u) * _L, _L)] = zeros16
        return carry

    lax.fori_loop(0, _BUF // (_L * 8), zero, 0)

    def chunk(g, carry):
        off = g * _C
        idx = row_off + p_v[pl.ds(off, _L)]
        plsc.store_scatter(buf, [idx], ones16)
        pltpu.sync_copy(buf, out_hbm.at[pl.ds((base + off) * N_CLASSES, _BUF)])
        plsc.store_scatter(buf, [idx], zeros16)
        return carry

    lax.fori_loop(0, _NCHUNK, chunk, 0)


def kernel(p, class_matrix):
    mesh = plsc.VectorSubcoreMesh(core_axis_name="c", subcore_axis_name="s")
    run = functools.partial(
        pl.kernel,
        mesh=mesh,
        out_type=jax.ShapeDtypeStruct((BATCH * N_CLASSES,), jnp.float32),
        scratch_types=[
            pltpu.VMEM((_ROWS_PER_W,), jnp.int32),
            pltpu.VMEM((_BUF,), jnp.float32),
        ],
        compiler_params=pltpu.CompilerParams(needs_layout_passes=False),
    )(_onehot_body)
    out = run(p.astype(jnp.int32), class_matrix)
    return out.reshape(BATCH, N_CLASSES)
